# baseline (speedup 1.0000x reference)
import functools

import jax
import jax.numpy as jnp
from jax import lax
from jax.experimental import pallas as pl
from jax.experimental.pallas import tpu as pltpu
from jax.experimental.pallas import tpu_sc as plsc

NC = 2
NS = 16
NW = NC * NS
L = 16

B = 16384
NB = 128
NCH = NB // L
ROWS_PER_W = B // NW
RBLK = 64
NBLK = ROWS_PER_W // RBLK


def _sc_body(bins_hbm, w_hbm, prm_hbm, out_hbm, bins_v, w_v, out_v, cdf_v,
             hist_v, prm_v):
    wid = lax.axis_index("s") * NC + lax.axis_index("c")
    row0 = wid * ROWS_PER_W

    pltpu.sync_copy(prm_hbm, prm_v)
    pv = prm_v[...]
    u0 = pv[0]
    step = pv[1]
    inv_step = pv[2]

    ii = lax.iota(jnp.int32, L)
    iif = ii.astype(jnp.float32)
    ones_i = jnp.ones((L,), jnp.int32)
    zeros_i = jnp.zeros((L,), jnp.int32)
    _gdn = lax.GatherDimensionNumbers(
        offset_dims=(), collapsed_slice_dims=(0,), start_index_map=(0,))
    _last = jnp.full((L,), L - 1, jnp.int32)

    def bcast15(x):
        return lax.gather(x, _last[:, None], _gdn, (1,),
                          mode=lax.GatherScatterMode.PROMISE_IN_BOUNDS)

    bias0 = jnp.where(ii == 0, 0.0, 1e-5).astype(jnp.float32)
    uvecs = [u0 + (iif + jnp.float32(c * L)) * step for c in range(NCH)]

    def do_row(r):
        rsplat = jnp.full((L,), r, jnp.int32)
        cs = []
        for c in range(NCH):
            wv = w_v[r, pl.ds(c * L, L)]
            wv = wv + (bias0 if c == 0 else jnp.float32(1e-5))
            cs.append(jnp.cumsum(wv))
        pre, carry = [], jnp.zeros((L,), jnp.float32)
        for c in range(NCH):
            pre.append(carry)
            carry = carry + bcast15(cs[c])
        inv_t = 1.0 / carry

        for c in range(NCH + 1):
            hist_v[r, pl.ds(c * L, L)] = zeros_i

        for c in range(NCH):
            csn = (cs[c] + pre[c]) * inv_t
            cdf_v[r, pl.ds(c * L, L)] = csn
            x = (csn - u0) * inv_step
            x = jnp.minimum(jnp.maximum(x, 0.0), jnp.float32(NB))
            ki = x.astype(jnp.int32)
            k = jnp.where(ki.astype(jnp.float32) < x, ki + 1, ki)
            plsc.addupdate_scatter(hist_v, [rsplat, k], ones_i)

        hcs = []
        for c in range(NCH):
            h = hist_v[r, pl.ds(c * L, L)]
            hcs.append(jnp.cumsum(h))
        icarry = jnp.zeros((L,), jnp.int32)
        for c in range(NCH):
            inds = hcs[c] + icarry
            icarry = icarry + bcast15(hcs[c])
            below = jnp.maximum(inds - 1, 0)
            above = jnp.minimum(inds, NB - 1)
            cdf_b = plsc.load_gather(cdf_v, [rsplat, below])
            cdf_a = plsc.load_gather(cdf_v, [rsplat, above])
            bins_b = plsc.load_gather(bins_v, [rsplat, below])
            bins_a = plsc.load_gather(bins_v, [rsplat, above])
            denom = cdf_a - cdf_b
            denom = jnp.where(denom < 1e-5, 1.0, denom)
            t = (uvecs[c] - cdf_b) / denom
            out_v[r, pl.ds(c * L, L)] = bins_b + t * (bins_a - bins_b)

    def blk_body(blk, _):
        base = row0 + blk * RBLK
        pltpu.sync_copy(bins_hbm.at[pl.ds(base, RBLK)], bins_v)
        pltpu.sync_copy(w_hbm.at[pl.ds(base, RBLK)], w_v)

        @plsc.parallel_loop(0, RBLK, 1, unroll=8)
        def _row(r):
            do_row(r)

        pltpu.sync_copy(out_v, out_hbm.at[pl.ds(base, RBLK)])
        return 0

    lax.fori_loop(0, NBLK, blk_body, 0)


@functools.partial(
    pl.kernel,
    out_type=jax.ShapeDtypeStruct((B, NB), jnp.float32),
    mesh=plsc.VectorSubcoreMesh(core_axis_name="c", subcore_axis_name="s",
                                num_cores=NC, num_subcores=NS),
    compiler_params=pltpu.CompilerParams(needs_layout_passes=False),
    scratch_types=[
        pltpu.VMEM((RBLK, NB), jnp.float32),
        pltpu.VMEM((RBLK, NB), jnp.float32),
        pltpu.VMEM((RBLK, NB), jnp.float32),
        pltpu.VMEM((RBLK, NB), jnp.float32),
        pltpu.VMEM((RBLK, NB + L), jnp.int32),
        pltpu.VMEM((L,), jnp.float32),
    ],
)
def _sample_pdf_sc(bins_hbm, w_hbm, prm_hbm, out_hbm, *scratch):
    _sc_body(bins_hbm, w_hbm, prm_hbm, out_hbm, *scratch)


def kernel(bins, weights, n_samples):
    n = jnp.asarray(n_samples, jnp.float32)
    ns = bins.shape[-1]
    u0 = 0.5 / n
    step = (1.0 - 1.0 / n) / jnp.float32(ns - 1)
    prm = jnp.zeros((L,), jnp.float32)
    prm = prm.at[0].set(u0).at[1].set(step).at[2].set(1.0 / step)
    wpad = jnp.concatenate(
        [jnp.zeros((bins.shape[0], 1), jnp.float32), weights], axis=1)
    return _sample_pdf_sc(bins, wpad, prm)


# speedup vs baseline: 2.1334x; 2.1334x over previous
import functools

import jax
import jax.numpy as jnp
from jax import lax
from jax.experimental import pallas as pl
from jax.experimental.pallas import tpu as pltpu
from jax.experimental.pallas import tpu_sc as plsc

NC = 2
NS = 16
NW = NC * NS
L = 16

B = 16384
NB = 128
NCH = NB // L
ROWS_PER_W = B // NW
RBLK = 64
NBLK = ROWS_PER_W // RBLK


def _sc_body(bins_hbm, w_hbm, prm_hbm, out_hbm, bins_v, w_v, out_v, cdf_v,
             hist_v, prm_v):
    wid = lax.axis_index("s") * NC + lax.axis_index("c")
    row0 = wid * ROWS_PER_W

    pltpu.sync_copy(prm_hbm, prm_v)
    pv = prm_v[...]
    u0 = pv[0]
    step = pv[1]
    inv_step = pv[2]

    ii = lax.iota(jnp.int32, L)
    iif = ii.astype(jnp.float32)
    ones_i = jnp.ones((L,), jnp.int32)
    zeros_i = jnp.zeros((L,), jnp.int32)
    _gdn = lax.GatherDimensionNumbers(
        offset_dims=(), collapsed_slice_dims=(0,), start_index_map=(0,))
    _last = jnp.full((L,), L - 1, jnp.int32)

    def bcast15(x):
        return lax.gather(x, _last[:, None], _gdn, (1,),
                          mode=lax.GatherScatterMode.PROMISE_IN_BOUNDS)

    bias0 = jnp.where(ii == 0, 0.0, 1e-5).astype(jnp.float32)
    uvecs = [u0 + (iif + jnp.float32(c * L)) * step for c in range(NCH)]

    def do_row(r):
        rsplat = jnp.full((L,), r, jnp.int32)
        cs = []
        for c in range(NCH):
            wv = w_v[r, pl.ds(c * L, L)]
            wv = wv + (bias0 if c == 0 else jnp.float32(1e-5))
            cs.append(jnp.cumsum(wv))
        pre, carry = [], jnp.zeros((L,), jnp.float32)
        for c in range(NCH):
            pre.append(carry)
            carry = carry + bcast15(cs[c])
        inv_t = 1.0 / carry

        for c in range(NCH + 1):
            hist_v[r, pl.ds(c * L, L)] = zeros_i

        for c in range(NCH):
            csn = (cs[c] + pre[c]) * inv_t
            cdf_v[r, pl.ds(c * L, L)] = csn
            x = (csn - u0) * inv_step
            x = jnp.minimum(jnp.maximum(x, 0.0), jnp.float32(NB))
            ki = x.astype(jnp.int32)
            k = jnp.where(ki.astype(jnp.float32) < x, ki + 1, ki)
            plsc.addupdate_scatter(hist_v, [rsplat, k], ones_i)

        hcs = []
        for c in range(NCH):
            h = hist_v[r, pl.ds(c * L, L)]
            hcs.append(jnp.cumsum(h))
        icarry = jnp.zeros((L,), jnp.int32)
        for c in range(NCH):
            inds = hcs[c] + icarry
            icarry = icarry + bcast15(hcs[c])
            below = jnp.maximum(inds - 1, 0)
            above = jnp.minimum(inds, NB - 1)
            cdf_b = plsc.load_gather(cdf_v, [rsplat, below])
            cdf_a = plsc.load_gather(cdf_v, [rsplat, above])
            bins_b = plsc.load_gather(bins_v, [rsplat, below])
            bins_a = plsc.load_gather(bins_v, [rsplat, above])
            denom = cdf_a - cdf_b
            denom = jnp.where(denom < 1e-5, 1.0, denom)
            t = (uvecs[c] - cdf_b) / denom
            out_v[r, pl.ds(c * L, L)] = bins_b + t * (bins_a - bins_b)

    def blk_body(blk, _):
        base = row0 + blk * RBLK
        pltpu.sync_copy(bins_hbm.at[pl.ds(base, RBLK)], bins_v)
        pltpu.sync_copy(w_hbm.at[pl.ds(base, RBLK)], w_v)

        @plsc.parallel_loop(0, RBLK, 1, unroll=2)
        def _row(r):
            do_row(r)

        pltpu.sync_copy(out_v, out_hbm.at[pl.ds(base, RBLK)])
        return 0

    lax.fori_loop(0, NBLK, blk_body, 0)


@functools.partial(
    pl.kernel,
    out_type=jax.ShapeDtypeStruct((B, NB), jnp.float32),
    mesh=plsc.VectorSubcoreMesh(core_axis_name="c", subcore_axis_name="s",
                                num_cores=NC, num_subcores=NS),
    compiler_params=pltpu.CompilerParams(needs_layout_passes=False),
    scratch_types=[
        pltpu.VMEM((RBLK, NB), jnp.float32),
        pltpu.VMEM((RBLK, NB), jnp.float32),
        pltpu.VMEM((RBLK, NB), jnp.float32),
        pltpu.VMEM((RBLK, NB), jnp.float32),
        pltpu.VMEM((RBLK, NB + L), jnp.int32),
        pltpu.VMEM((L,), jnp.float32),
    ],
)
def _sample_pdf_sc(bins_hbm, w_hbm, prm_hbm, out_hbm, *scratch):
    _sc_body(bins_hbm, w_hbm, prm_hbm, out_hbm, *scratch)


def kernel(bins, weights, n_samples):
    n = jnp.asarray(n_samples, jnp.float32)
    ns = bins.shape[-1]
    u0 = 0.5 / n
    step = (1.0 - 1.0 / n) / jnp.float32(ns - 1)
    prm = jnp.zeros((L,), jnp.float32)
    prm = prm.at[0].set(u0).at[1].set(step).at[2].set(1.0 / step)
    wpad = jnp.concatenate(
        [jnp.zeros((bins.shape[0], 1), jnp.float32), weights], axis=1)
    return _sample_pdf_sc(bins, wpad, prm)


# speedup vs baseline: 2.5081x; 1.1757x over previous
import functools

import jax
import jax.numpy as jnp
from jax import lax
from jax.experimental import pallas as pl
from jax.experimental.pallas import tpu as pltpu
from jax.experimental.pallas import tpu_sc as plsc

NC = 2
NS = 16
NW = NC * NS
L = 16

B = 16384
NB = 128
NCH = NB // L
ROWS_PER_W = B // NW
RBLK = 64
NBLK = ROWS_PER_W // RBLK


def _sc_body(bins_hbm, w_hbm, prm_hbm, out_hbm, bins_v, w_v, out_v, cdf_v,
             hist_v, prm_v):
    wid = lax.axis_index("s") * NC + lax.axis_index("c")
    row0 = wid * ROWS_PER_W

    pltpu.sync_copy(prm_hbm, prm_v)
    pv = prm_v[...]
    u0 = pv[0]
    step = pv[1]
    inv_step = pv[2]

    ii = lax.iota(jnp.int32, L)
    iif = ii.astype(jnp.float32)
    ones_i = jnp.ones((L,), jnp.int32)
    zeros_i = jnp.zeros((L,), jnp.int32)
    _gdn = lax.GatherDimensionNumbers(
        offset_dims=(), collapsed_slice_dims=(0,), start_index_map=(0,))
    _last = jnp.full((L,), L - 1, jnp.int32)

    def bcast15(x):
        return lax.gather(x, _last[:, None], _gdn, (1,),
                          mode=lax.GatherScatterMode.PROMISE_IN_BOUNDS)

    bias0 = jnp.where(ii == 0, 0.0, 1e-5).astype(jnp.float32)
    uvecs = [u0 + (iif + jnp.float32(c * L)) * step for c in range(NCH)]

    def do_row(r):
        rsplat = jnp.full((L,), r, jnp.int32)
        cs = []
        for c in range(NCH):
            wv = w_v[r, pl.ds(c * L, L)]
            wv = wv + (bias0 if c == 0 else jnp.float32(1e-5))
            cs.append(jnp.cumsum(wv))
        pre, carry = [], jnp.zeros((L,), jnp.float32)
        for c in range(NCH):
            pre.append(carry)
            carry = carry + bcast15(cs[c])
        inv_t = 1.0 / carry

        for c in range(NCH + 1):
            hist_v[r, pl.ds(c * L, L)] = zeros_i

        for c in range(NCH):
            csn = (cs[c] + pre[c]) * inv_t
            cdf_v[r, pl.ds(c * L, L)] = csn
            x = (csn - u0) * inv_step
            x = jnp.minimum(jnp.maximum(x, 0.0), jnp.float32(NB))
            ki = x.astype(jnp.int32)
            k = jnp.where(ki.astype(jnp.float32) < x, ki + 1, ki)
            plsc.addupdate_scatter(hist_v, [rsplat, k], ones_i)

        hcs = []
        for c in range(NCH):
            h = hist_v[r, pl.ds(c * L, L)]
            hcs.append(jnp.cumsum(h))
        icarry = jnp.zeros((L,), jnp.int32)
        for c in range(NCH):
            inds = hcs[c] + icarry
            icarry = icarry + bcast15(hcs[c])
            below = jnp.maximum(inds - 1, 0)
            above = jnp.minimum(inds, NB - 1)
            cdf_b = plsc.load_gather(cdf_v, [rsplat, below])
            cdf_a = plsc.load_gather(cdf_v, [rsplat, above])
            bins_b = plsc.load_gather(bins_v, [rsplat, below])
            bins_a = plsc.load_gather(bins_v, [rsplat, above])
            denom = cdf_a - cdf_b
            denom = jnp.where(denom < 1e-5, 1.0, denom)
            t = (uvecs[c] - cdf_b) / denom
            out_v[r, pl.ds(c * L, L)] = bins_b + t * (bins_a - bins_b)

    def blk_body(blk, _):
        base = row0 + blk * RBLK
        pltpu.sync_copy(bins_hbm.at[pl.ds(base, RBLK)], bins_v)
        pltpu.sync_copy(w_hbm.at[pl.ds(base, RBLK)], w_v)

        @plsc.parallel_loop(0, RBLK, 1, unroll=1)
        def _row(r):
            do_row(r)

        pltpu.sync_copy(out_v, out_hbm.at[pl.ds(base, RBLK)])
        return 0

    lax.fori_loop(0, NBLK, blk_body, 0)


@functools.partial(
    pl.kernel,
    out_type=jax.ShapeDtypeStruct((B, NB), jnp.float32),
    mesh=plsc.VectorSubcoreMesh(core_axis_name="c", subcore_axis_name="s",
                                num_cores=NC, num_subcores=NS),
    compiler_params=pltpu.CompilerParams(needs_layout_passes=False),
    scratch_types=[
        pltpu.VMEM((RBLK, NB), jnp.float32),
        pltpu.VMEM((RBLK, NB), jnp.float32),
        pltpu.VMEM((RBLK, NB), jnp.float32),
        pltpu.VMEM((RBLK, NB), jnp.float32),
        pltpu.VMEM((RBLK, NB + L), jnp.int32),
        pltpu.VMEM((L,), jnp.float32),
    ],
)
def _sample_pdf_sc(bins_hbm, w_hbm, prm_hbm, out_hbm, *scratch):
    _sc_body(bins_hbm, w_hbm, prm_hbm, out_hbm, *scratch)


def kernel(bins, weights, n_samples):
    n = jnp.asarray(n_samples, jnp.float32)
    ns = bins.shape[-1]
    u0 = 0.5 / n
    step = (1.0 - 1.0 / n) / jnp.float32(ns - 1)
    prm = jnp.zeros((L,), jnp.float32)
    prm = prm.at[0].set(u0).at[1].set(step).at[2].set(1.0 / step)
    wpad = jnp.concatenate(
        [jnp.zeros((bins.shape[0], 1), jnp.float32), weights], axis=1)
    return _sample_pdf_sc(bins, wpad, prm)


# speedup vs baseline: 2.6852x; 1.0706x over previous
import functools

import jax
import jax.numpy as jnp
from jax import lax
from jax.experimental import pallas as pl
from jax.experimental.pallas import tpu as pltpu
from jax.experimental.pallas import tpu_sc as plsc

NC = 2
NS = 16
NW = NC * NS
L = 16

B = 16384
NB = 128
NCH = NB // L
ROWS_PER_W = B // NW
RBLK = 128
NBLK = ROWS_PER_W // RBLK


def _sc_body(bins_hbm, w_hbm, prm_hbm, out_hbm, bins_v, w_v, out_v, cdf_v,
             hist_v, prm_v):
    wid = lax.axis_index("s") * NC + lax.axis_index("c")
    row0 = wid * ROWS_PER_W

    pltpu.sync_copy(prm_hbm, prm_v)
    pv = prm_v[...]
    u0 = pv[0]
    step = pv[1]
    inv_step = pv[2]

    ii = lax.iota(jnp.int32, L)
    iif = ii.astype(jnp.float32)
    ones_i = jnp.ones((L,), jnp.int32)
    zeros_i = jnp.zeros((L,), jnp.int32)
    _gdn = lax.GatherDimensionNumbers(
        offset_dims=(), collapsed_slice_dims=(0,), start_index_map=(0,))
    _last = jnp.full((L,), L - 1, jnp.int32)

    def bcast15(x):
        return lax.gather(x, _last[:, None], _gdn, (1,),
                          mode=lax.GatherScatterMode.PROMISE_IN_BOUNDS)

    bias0 = jnp.where(ii == 0, 0.0, 1e-5).astype(jnp.float32)
    uvecs = [u0 + (iif + jnp.float32(c * L)) * step for c in range(NCH)]

    def do_row(r):
        rsplat = jnp.full((L,), r, jnp.int32)
        cs = []
        for c in range(NCH):
            wv = w_v[r, pl.ds(c * L, L)]
            wv = wv + (bias0 if c == 0 else jnp.float32(1e-5))
            cs.append(jnp.cumsum(wv))
        pre, carry = [], jnp.zeros((L,), jnp.float32)
        for c in range(NCH):
            pre.append(carry)
            carry = carry + bcast15(cs[c])
        inv_t = 1.0 / carry

        for c in range(NCH + 1):
            hist_v[r, pl.ds(c * L, L)] = zeros_i

        for c in range(NCH):
            csn = (cs[c] + pre[c]) * inv_t
            cdf_v[r, pl.ds(c * L, L)] = csn
            x = (csn - u0) * inv_step
            x = jnp.minimum(jnp.maximum(x, 0.0), jnp.float32(NB))
            ki = x.astype(jnp.int32)
            k = jnp.where(ki.astype(jnp.float32) < x, ki + 1, ki)
            plsc.addupdate_scatter(hist_v, [rsplat, k], ones_i)

        hcs = []
        for c in range(NCH):
            h = hist_v[r, pl.ds(c * L, L)]
            hcs.append(jnp.cumsum(h))
        icarry = jnp.zeros((L,), jnp.int32)
        for c in range(NCH):
            inds = hcs[c] + icarry
            icarry = icarry + bcast15(hcs[c])
            below = jnp.maximum(inds - 1, 0)
            above = jnp.minimum(inds, NB - 1)
            cdf_b = plsc.load_gather(cdf_v, [rsplat, below])
            cdf_a = plsc.load_gather(cdf_v, [rsplat, above])
            bins_b = plsc.load_gather(bins_v, [rsplat, below])
            bins_a = plsc.load_gather(bins_v, [rsplat, above])
            denom = cdf_a - cdf_b
            denom = jnp.where(denom < 1e-5, 1.0, denom)
            t = (uvecs[c] - cdf_b) / denom
            out_v[r, pl.ds(c * L, L)] = bins_b + t * (bins_a - bins_b)

    def blk_body(blk, _):
        base = row0 + blk * RBLK
        pltpu.sync_copy(bins_hbm.at[pl.ds(base, RBLK)], bins_v)
        pltpu.sync_copy(w_hbm.at[pl.ds(base, RBLK)], w_v)

        @plsc.parallel_loop(0, RBLK, 1, unroll=1)
        def _row(r):
            do_row(r)

        pltpu.sync_copy(out_v, out_hbm.at[pl.ds(base, RBLK)])
        return 0

    lax.fori_loop(0, NBLK, blk_body, 0)


@functools.partial(
    pl.kernel,
    out_type=jax.ShapeDtypeStruct((B, NB), jnp.float32),
    mesh=plsc.VectorSubcoreMesh(core_axis_name="c", subcore_axis_name="s",
                                num_cores=NC, num_subcores=NS),
    compiler_params=pltpu.CompilerParams(needs_layout_passes=False),
    scratch_types=[
        pltpu.VMEM((RBLK, NB), jnp.float32),
        pltpu.VMEM((RBLK, NB), jnp.float32),
        pltpu.VMEM((RBLK, NB), jnp.float32),
        pltpu.VMEM((RBLK, NB), jnp.float32),
        pltpu.VMEM((RBLK, NB + L), jnp.int32),
        pltpu.VMEM((L,), jnp.float32),
    ],
)
def _sample_pdf_sc(bins_hbm, w_hbm, prm_hbm, out_hbm, *scratch):
    _sc_body(bins_hbm, w_hbm, prm_hbm, out_hbm, *scratch)


def kernel(bins, weights, n_samples):
    n = jnp.asarray(n_samples, jnp.float32)
    ns = bins.shape[-1]
    u0 = 0.5 / n
    step = (1.0 - 1.0 / n) / jnp.float32(ns - 1)
    prm = jnp.zeros((L,), jnp.float32)
    prm = prm.at[0].set(u0).at[1].set(step).at[2].set(1.0 / step)
    wpad = jnp.concatenate(
        [jnp.zeros((bins.shape[0], 1), jnp.float32), weights], axis=1)
    return _sample_pdf_sc(bins, wpad, prm)


# speedup vs baseline: 2.8180x; 1.0495x over previous
import functools

import jax
import jax.numpy as jnp
from jax import lax
from jax.experimental import pallas as pl
from jax.experimental.pallas import tpu as pltpu
from jax.experimental.pallas import tpu_sc as plsc

NC = 2
NS = 16
NW = NC * NS
L = 16

B = 16384
NB = 128
NCH = NB // L
ROWS_PER_W = B // NW
RBLK = 64
NBLK = ROWS_PER_W // RBLK


def _sc_body(bins_hbm, w_hbm, prm_hbm, out_hbm, bins_v0, w_v0, out_v0,
             bins_v1, w_v1, out_v1, cdf_v, hist_v, prm_v, sb0, sw0, so0,
             sb1, sw1, so1):
    wid = lax.axis_index("s") * NC + lax.axis_index("c")
    row0 = wid * ROWS_PER_W

    pltpu.sync_copy(prm_hbm, prm_v)
    pv = prm_v[...]
    u0 = pv[0]
    step = pv[1]
    inv_step = pv[2]

    ii = lax.iota(jnp.int32, L)
    iif = ii.astype(jnp.float32)
    ones_i = jnp.ones((L,), jnp.int32)
    zeros_i = jnp.zeros((L,), jnp.int32)
    _gdn = lax.GatherDimensionNumbers(
        offset_dims=(), collapsed_slice_dims=(0,), start_index_map=(0,))
    _last = jnp.full((L,), L - 1, jnp.int32)

    def bcast15(x):
        return lax.gather(x, _last[:, None], _gdn, (1,),
                          mode=lax.GatherScatterMode.PROMISE_IN_BOUNDS)

    bias0 = jnp.where(ii == 0, 0.0, 1e-5).astype(jnp.float32)
    uvecs = [u0 + (iif + jnp.float32(c * L)) * step for c in range(NCH)]

    def do_row(r, bins_v, w_v, out_v):
        rsplat = jnp.full((L,), r, jnp.int32)
        cs = []
        for c in range(NCH):
            wv = w_v[r, pl.ds(c * L, L)]
            wv = wv + (bias0 if c == 0 else jnp.float32(1e-5))
            cs.append(jnp.cumsum(wv))
        pre, carry = [], jnp.zeros((L,), jnp.float32)
        for c in range(NCH):
            pre.append(carry)
            carry = carry + bcast15(cs[c])
        inv_t = 1.0 / carry

        for c in range(NCH + 1):
            hist_v[r, pl.ds(c * L, L)] = zeros_i

        for c in range(NCH):
            csn = (cs[c] + pre[c]) * inv_t
            cdf_v[r, pl.ds(c * L, L)] = csn
            x = (csn - u0) * inv_step
            x = jnp.minimum(jnp.maximum(x, 0.0), jnp.float32(NB))
            ki = x.astype(jnp.int32)
            k = jnp.where(ki.astype(jnp.float32) < x, ki + 1, ki)
            plsc.addupdate_scatter(hist_v, [rsplat, k], ones_i)

        hcs = []
        for c in range(NCH):
            h = hist_v[r, pl.ds(c * L, L)]
            hcs.append(jnp.cumsum(h))
        icarry = jnp.zeros((L,), jnp.int32)
        for c in range(NCH):
            inds = hcs[c] + icarry
            icarry = icarry + bcast15(hcs[c])
            below = jnp.maximum(inds - 1, 0)
            above = jnp.minimum(inds, NB - 1)
            cdf_b = plsc.load_gather(cdf_v, [rsplat, below])
            cdf_a = plsc.load_gather(cdf_v, [rsplat, above])
            bins_b = plsc.load_gather(bins_v, [rsplat, below])
            bins_a = plsc.load_gather(bins_v, [rsplat, above])
            denom = cdf_a - cdf_b
            denom = jnp.where(denom < 1e-5, 1.0, denom)
            t = (uvecs[c] - cdf_b) / denom
            out_v[r, pl.ds(c * L, L)] = bins_b + t * (bins_a - bins_b)

    bufs = [(bins_v0, w_v0, out_v0, sb0, sw0, so0),
            (bins_v1, w_v1, out_v1, sb1, sw1, so1)]

    def start_in(blk, par):
        base = row0 + blk * RBLK
        bv, wv, _, sb, sw, _ = bufs[par]
        pltpu.async_copy(bins_hbm.at[pl.ds(base, RBLK)], bv, sb)
        pltpu.async_copy(w_hbm.at[pl.ds(base, RBLK)], wv, sw)

    start_in(0, 0)
    for blk in range(NBLK):
        par = blk & 1
        base = row0 + blk * RBLK
        bv, wv, ov, sb, sw, so = bufs[par]
        if blk + 1 < NBLK:
            start_in(blk + 1, 1 - par)
        pltpu.make_async_copy(bins_hbm.at[pl.ds(base, RBLK)], bv, sb).wait()
        pltpu.make_async_copy(w_hbm.at[pl.ds(base, RBLK)], wv, sw).wait()
        if blk >= 2:
            pbase = row0 + (blk - 2) * RBLK
            pltpu.make_async_copy(
                ov, out_hbm.at[pl.ds(pbase, RBLK)], so).wait()

        @plsc.parallel_loop(0, RBLK, 1, unroll=1)
        def _row(r, _bv=bv, _wv=wv, _ov=ov):
            do_row(r, _bv, _wv, _ov)

        pltpu.async_copy(ov, out_hbm.at[pl.ds(base, RBLK)], so)
    for blk in (NBLK - 2, NBLK - 1):
        par = blk & 1
        base = row0 + blk * RBLK
        ov, so = bufs[par][2], bufs[par][5]
        pltpu.make_async_copy(ov, out_hbm.at[pl.ds(base, RBLK)], so).wait()


@functools.partial(
    pl.kernel,
    out_type=jax.ShapeDtypeStruct((B, NB), jnp.float32),
    mesh=plsc.VectorSubcoreMesh(core_axis_name="c", subcore_axis_name="s",
                                num_cores=NC, num_subcores=NS),
    compiler_params=pltpu.CompilerParams(needs_layout_passes=False),
    scratch_types=[
        pltpu.VMEM((RBLK, NB), jnp.float32),
        pltpu.VMEM((RBLK, NB), jnp.float32),
        pltpu.VMEM((RBLK, NB), jnp.float32),
        pltpu.VMEM((RBLK, NB), jnp.float32),
        pltpu.VMEM((RBLK, NB), jnp.float32),
        pltpu.VMEM((RBLK, NB), jnp.float32),
        pltpu.VMEM((RBLK, NB), jnp.float32),
        pltpu.VMEM((RBLK, NB + L), jnp.int32),
        pltpu.VMEM((L,), jnp.float32),
        pltpu.SemaphoreType.DMA,
        pltpu.SemaphoreType.DMA,
        pltpu.SemaphoreType.DMA,
        pltpu.SemaphoreType.DMA,
        pltpu.SemaphoreType.DMA,
        pltpu.SemaphoreType.DMA,
    ],
)
def _sample_pdf_sc(bins_hbm, w_hbm, prm_hbm, out_hbm, *scratch):
    _sc_body(bins_hbm, w_hbm, prm_hbm, out_hbm, *scratch)


def kernel(bins, weights, n_samples):
    n = jnp.asarray(n_samples, jnp.float32)
    ns = bins.shape[-1]
    u0 = 0.5 / n
    step = (1.0 - 1.0 / n) / jnp.float32(ns - 1)
    prm = jnp.zeros((L,), jnp.float32)
    prm = prm.at[0].set(u0).at[1].set(step).at[2].set(1.0 / step)
    wpad = jnp.concatenate(
        [jnp.zeros((bins.shape[0], 1), jnp.float32), weights], axis=1)
    return _sample_pdf_sc(bins, wpad, prm)
